# baseline (speedup 1.0000x reference)
import jax
import jax.numpy as jnp
from jax import lax
from jax.experimental import pallas as pl
from jax.experimental.pallas import tpu as pltpu
from jax.experimental.pallas import tpu_sc as plsc

_B, _S, _H, _F, _E, _K = 1, 2048, 1024, 2816, 8, 2
_T = _B * _S
_P = _T * _K
_BLK = 256
_NB = _P // _BLK + _E
_PP = _NB * _BLK
_FT = 1408
_NF = _F // _FT

_NC, _NS = 2, 16
_NW = _NC * _NS
_CTOK = _T // _NW
_CCH = 32


def _router_kernel(hs_ref, rw_ref, logits_ref, w_ref, e_ref):
    hs = hs_ref[...]
    logits = jax.lax.dot_general(hs, rw_ref[...], (((1,), (1,)), ((), ())),
                                 preferred_element_type=jnp.float32)
    logits_ref[...] = logits
    m = jnp.max(logits, axis=1, keepdims=True)
    ex = jnp.exp(logits - m)
    sm = ex / jnp.sum(ex, axis=1, keepdims=True)
    iota = jax.lax.broadcasted_iota(jnp.int32, sm.shape, 1)
    m1 = jnp.max(sm, axis=1, keepdims=True)
    a1 = jnp.min(jnp.where(sm == m1, iota, _E), axis=1, keepdims=True)
    sm2 = jnp.where(iota == a1, -jnp.inf, sm)
    m2 = jnp.max(sm2, axis=1, keepdims=True)
    a2 = jnp.min(jnp.where(sm2 == m2, iota, _E), axis=1, keepdims=True)
    w_ref[...] = jnp.concatenate([m1, m2], axis=1)
    e_ref[...] = jnp.concatenate([a1, a2], axis=1)


def _sc_combine_body(ys_ref, p0_ref, p1_ref, out_ref, i0_v, i1_v, r0_v, r1_v,
                     sem):
    wid = lax.axis_index("s") * _NC + lax.axis_index("c")
    base = wid * _CTOK
    for ch in range(_CTOK // _CCH):
        off = base + ch * _CCH
        pltpu.sync_copy(p0_ref.at[pl.ds(off, _CCH)], i0_v)
        pltpu.sync_copy(p1_ref.at[pl.ds(off, _CCH)], i1_v)
        pltpu.async_copy(ys_ref.at[i0_v], r0_v, sem).wait()
        pltpu.async_copy(ys_ref.at[i1_v], r1_v, sem).wait()
        for i in range(_CCH):
            def add16(j, c):
                sl = pl.ds(j * 16, 16)
                r0_v[i, sl] = r0_v[i, sl] + r1_v[i, sl]
                return c
            lax.fori_loop(0, _H // 16, add16, 0, unroll=8)
        pltpu.sync_copy(r0_v, out_ref.at[pl.ds(off, _CCH)])


def _gather_issue(rt_ref, hs_ref, dst, sem, bb):
    def issue(i, c):
        t = rt_ref[bb * _BLK + i]
        pltpu.make_async_copy(hs_ref.at[pl.ds(t, 1), :],
                              dst.at[pl.ds(i, 1), :], sem).start()
        return c
    jax.lax.fori_loop(0, _BLK, issue, 0, unroll=8)


def _gather_wait(hs_ref, dst, sem):
    def w(i, c):
        pltpu.make_async_copy(hs_ref.at[pl.ds(0, 1), :],
                              dst.at[pl.ds(i, 1), :], sem).wait()
        return c
    jax.lax.fori_loop(0, _BLK, w, 0, unroll=8)


def _ffn_kernel(be_ref, rt_ref, ba_ref, hs_ref, g_ref, u_ref, d_ref, w_ref,
                ys_ref, xb0, xb1, xc, acc, sem):
    b = pl.program_id(0)
    f = pl.program_id(1)
    active = ba_ref[b] == 1

    @pl.when(active)
    def _():
        @pl.when(f == 0)
        def _():
            @pl.when(b == 0)
            def _():
                _gather_issue(rt_ref, hs_ref, xb0, sem, 0)

            @pl.when(b % 2 == 0)
            def _():
                _gather_wait(hs_ref, xb0, sem)
                xc[...] = xb0[...]

            @pl.when(b % 2 == 1)
            def _():
                _gather_wait(hs_ref, xb1, sem)
                xc[...] = xb1[...]

            nxt = jnp.minimum(b + 1, _NB - 1)
            nxt_active = ((b + 1 < _NB) & (ba_ref[nxt] == 1))

            @pl.when(nxt_active & (b % 2 == 0))
            def _():
                _gather_issue(rt_ref, hs_ref, xb1, sem, nxt)

            @pl.when(nxt_active & (b % 2 == 1))
            def _():
                _gather_issue(rt_ref, hs_ref, xb0, sem, nxt)

        x = xc[...]
        g = jax.lax.dot_general(x, g_ref[0], (((1,), (1,)), ((), ())),
                                preferred_element_type=jnp.float32)
        u = jax.lax.dot_general(x, u_ref[0], (((1,), (1,)), ((), ())),
                                preferred_element_type=jnp.float32)
        h = (g * jax.nn.sigmoid(g)) * u
        part = jax.lax.dot_general(h, d_ref[0], (((1,), (1,)), ((), ())),
                                   preferred_element_type=jnp.float32)

        @pl.when(f == 0)
        def _():
            acc[...] = part

        @pl.when(f != 0)
        def _():
            acc[...] += part

        @pl.when(f == _NF - 1)
        def _():
            ys_ref[...] = acc[...] * w_ref[...]


def kernel(hidden_states, router_w, gate_w, up_w, down_w):
    b, s, h = hidden_states.shape
    hs = hidden_states.reshape(-1, h)

    logits, tw, te = pl.pallas_call(
        _router_kernel,
        out_shape=[
            jax.ShapeDtypeStruct((_T, _E), jnp.float32),
            jax.ShapeDtypeStruct((_T, _K), jnp.float32),
            jax.ShapeDtypeStruct((_T, _K), jnp.int32),
        ],
    )(hs, router_w)

    es = te.reshape(-1)
    ws = tw.reshape(-1)
    onehot = (es[:, None] == jnp.arange(_E)[None, :]).astype(jnp.int32)
    within = jnp.cumsum(onehot, axis=0) - onehot
    rank = jnp.sum(within * onehot, axis=1)
    counts = jnp.sum(onehot, axis=0)
    padded = ((counts + _BLK - 1) // _BLK) * _BLK
    pend = jnp.cumsum(padded)
    poff = pend - padded
    pos = poff[es] + rank
    row_token = jnp.zeros((_PP,), jnp.int32).at[pos].set(
        jnp.arange(_P, dtype=jnp.int32) // _K)
    row_weight = jnp.zeros((_PP, 1), jnp.float32).at[pos, 0].set(ws)
    pos2 = pos.reshape(_T, _K)
    pos0 = pos2[:, 0].astype(jnp.int32)
    pos1 = pos2[:, 1].astype(jnp.int32)
    starts = jnp.arange(_NB, dtype=jnp.int32) * _BLK
    total = pend[_E - 1]
    block_active = (starts < total).astype(jnp.int32)
    starts_c = jnp.minimum(starts, total - 1)
    block_expert = jnp.searchsorted(pend, starts_c, side='right').astype(jnp.int32)

    def _fsnake(bb, ff):
        return jnp.where(bb % 2 == 0, ff, _NF - 1 - ff)

    grid_spec = pltpu.PrefetchScalarGridSpec(
        num_scalar_prefetch=3,
        grid=(_NB, _NF),
        in_specs=[
            pl.BlockSpec(memory_space=pl.ANY),
            pl.BlockSpec((1, _FT, _H),
                         lambda bb, ff, be, rt, ba: (be[bb], _fsnake(bb, ff), 0)),
            pl.BlockSpec((1, _FT, _H),
                         lambda bb, ff, be, rt, ba: (be[bb], _fsnake(bb, ff), 0)),
            pl.BlockSpec((1, _H, _FT),
                         lambda bb, ff, be, rt, ba: (be[bb], 0, _fsnake(bb, ff))),
            pl.BlockSpec((_BLK, 1), lambda bb, ff, be, rt, ba: (bb, 0)),
        ],
        out_specs=pl.BlockSpec((_BLK, _H), lambda bb, ff, be, rt, ba: (bb, 0)),
        scratch_shapes=[
            pltpu.VMEM((_BLK, _H), jnp.float32),
            pltpu.VMEM((_BLK, _H), jnp.float32),
            pltpu.VMEM((_BLK, _H), jnp.float32),
            pltpu.VMEM((_BLK, _H), jnp.float32),
            pltpu.SemaphoreType.DMA,
        ],
    )
    ys = pl.pallas_call(
        _ffn_kernel,
        grid_spec=grid_spec,
        out_shape=jax.ShapeDtypeStruct((_PP, _H), jnp.float32),
        compiler_params=pltpu.CompilerParams(
            dimension_semantics=("arbitrary", "arbitrary"),
        ),
    )(block_expert, row_token, block_active,
      hs, gate_w, up_w, down_w, row_weight)

    mesh = plsc.VectorSubcoreMesh(core_axis_name="c", subcore_axis_name="s",
                                  num_cores=_NC, num_subcores=_NS)
    out = pl.kernel(
        _sc_combine_body,
        mesh=mesh,
        out_type=jax.ShapeDtypeStruct((_T, _H), jnp.float32),
        scratch_types=[
            pltpu.VMEM((_CCH,), jnp.int32),
            pltpu.VMEM((_CCH,), jnp.int32),
            pltpu.VMEM((_CCH, _H), jnp.float32),
            pltpu.VMEM((_CCH, _H), jnp.float32),
            pltpu.SemaphoreType.DMA,
        ],
    )(ys, pos0, pos1)

    return out.reshape(b, s, h), logits


# speedup vs baseline: 1.0978x; 1.0978x over previous
import jax
import jax.numpy as jnp
from jax import lax
from jax.experimental import pallas as pl
from jax.experimental.pallas import tpu as pltpu
from jax.experimental.pallas import tpu_sc as plsc

_B, _S, _H, _F, _E, _K = 1, 2048, 1024, 2816, 8, 2
_T = _B * _S
_P = _T * _K
_BLK = 256
_NB = _P // _BLK + _E
_PP = _NB * _BLK
_FT = 1408
_NF = _F // _FT

_NC, _NS = 2, 16
_NW = _NC * _NS
_CTOK = _T // _NW
_CCH = 16
_NCH = _CTOK // _CCH


def _router_kernel(hs_ref, rw_ref, logits_ref, w_ref, e_ref):
    hs = hs_ref[...]
    logits = jax.lax.dot_general(hs, rw_ref[...], (((1,), (1,)), ((), ())),
                                 preferred_element_type=jnp.float32)
    logits_ref[...] = logits
    m = jnp.max(logits, axis=1, keepdims=True)
    ex = jnp.exp(logits - m)
    sm = ex / jnp.sum(ex, axis=1, keepdims=True)
    iota = jax.lax.broadcasted_iota(jnp.int32, sm.shape, 1)
    m1 = jnp.max(sm, axis=1, keepdims=True)
    a1 = jnp.min(jnp.where(sm == m1, iota, _E), axis=1, keepdims=True)
    sm2 = jnp.where(iota == a1, -jnp.inf, sm)
    m2 = jnp.max(sm2, axis=1, keepdims=True)
    a2 = jnp.min(jnp.where(sm2 == m2, iota, _E), axis=1, keepdims=True)
    w_ref[...] = jnp.concatenate([m1, m2], axis=1)
    e_ref[...] = jnp.concatenate([a1, a2], axis=1)


def _sc_combine_body(ys_ref, p0_ref, p1_ref, out_ref, i0_v, i1_v,
                     r0a, r1a, r0b, r1b, sem):
    wid = lax.axis_index("s") * _NC + lax.axis_index("c")
    base = wid * _CTOK
    pltpu.sync_copy(p0_ref.at[pl.ds(base, _CTOK)], i0_v)
    pltpu.sync_copy(p1_ref.at[pl.ds(base, _CTOK)], i1_v)
    bufs = [(r0a, r1a), (r0b, r1b)]
    pend = [None, None]

    def start(ch):
        r0, r1 = bufs[ch % 2]
        c0 = pltpu.async_copy(ys_ref.at[i0_v.at[pl.ds(ch * _CCH, _CCH)]],
                              r0, sem)
        c1 = pltpu.async_copy(ys_ref.at[i1_v.at[pl.ds(ch * _CCH, _CCH)]],
                              r1, sem)
        pend[ch % 2] = (c0, c1)

    start(0)
    for ch in range(_NCH):
        if ch + 1 < _NCH:
            start(ch + 1)
        c0, c1 = pend[ch % 2]
        c0.wait()
        c1.wait()
        r0, r1 = bufs[ch % 2]

        def add_row(i, c):
            def add16(j, c2):
                sl = pl.ds(j * 16, 16)
                r0[i, sl] = r0[i, sl] + r1[i, sl]
                return c2
            lax.fori_loop(0, _H // 16, add16, 0, unroll=8)
            return c
        lax.fori_loop(0, _CCH, add_row, 0)
        pltpu.sync_copy(r0, out_ref.at[pl.ds(base + ch * _CCH, _CCH)])


def _ffn_kernel(be_ref, rt_ref, ba_ref, hs_ref, g_ref, u_ref, d_ref, w_ref,
                ys_ref, x_s, acc):
    b = pl.program_id(0)
    f = pl.program_id(1)
    active = ba_ref[b] == 1

    @pl.when(active)
    def _():
        @pl.when(f == 0)
        def _():
            def gather(i, c):
                t = rt_ref[b * _BLK + i]
                x_s[pl.ds(i, 1), :] = hs_ref[pl.ds(t, 1), :]
                return c
            jax.lax.fori_loop(0, _BLK, gather, 0, unroll=8)

        x = x_s[...]
        g = jax.lax.dot_general(x, g_ref[0], (((1,), (1,)), ((), ())),
                                preferred_element_type=jnp.float32)
        u = jax.lax.dot_general(x, u_ref[0], (((1,), (1,)), ((), ())),
                                preferred_element_type=jnp.float32)
        h = (g * jax.nn.sigmoid(g)) * u
        part = jax.lax.dot_general(h, d_ref[0], (((1,), (1,)), ((), ())),
                                   preferred_element_type=jnp.float32)

        @pl.when(f == 0)
        def _():
            acc[...] = part

        @pl.when(f != 0)
        def _():
            acc[...] += part

        @pl.when(f == _NF - 1)
        def _():
            ys_ref[...] = acc[...] * w_ref[...]


def kernel(hidden_states, router_w, gate_w, up_w, down_w):
    b, s, h = hidden_states.shape
    hs = hidden_states.reshape(-1, h)

    logits, tw, te = pl.pallas_call(
        _router_kernel,
        out_shape=[
            jax.ShapeDtypeStruct((_T, _E), jnp.float32),
            jax.ShapeDtypeStruct((_T, _K), jnp.float32),
            jax.ShapeDtypeStruct((_T, _K), jnp.int32),
        ],
    )(hs, router_w)

    es = te.reshape(-1)
    ws = tw.reshape(-1)
    onehot = (es[:, None] == jnp.arange(_E)[None, :]).astype(jnp.int32)
    within = jnp.cumsum(onehot, axis=0) - onehot
    rank = jnp.sum(within * onehot, axis=1)
    counts = jnp.sum(onehot, axis=0)
    padded = ((counts + _BLK - 1) // _BLK) * _BLK
    pend = jnp.cumsum(padded)
    poff = pend - padded
    pos = poff[es] + rank
    row_token = jnp.zeros((_PP,), jnp.int32).at[pos].set(
        jnp.arange(_P, dtype=jnp.int32) // _K)
    row_weight = jnp.zeros((_PP, 1), jnp.float32).at[pos, 0].set(ws)
    pos2 = pos.reshape(_T, _K)
    pos0 = pos2[:, 0].astype(jnp.int32)
    pos1 = pos2[:, 1].astype(jnp.int32)
    starts = jnp.arange(_NB, dtype=jnp.int32) * _BLK
    total = pend[_E - 1]
    block_active = (starts < total).astype(jnp.int32)
    starts_c = jnp.minimum(starts, total - 1)
    block_expert = jnp.searchsorted(pend, starts_c, side='right').astype(jnp.int32)

    def _fsnake(bb, ff):
        return jnp.where(bb % 2 == 0, ff, _NF - 1 - ff)

    grid_spec = pltpu.PrefetchScalarGridSpec(
        num_scalar_prefetch=3,
        grid=(_NB, _NF),
        in_specs=[
            pl.BlockSpec((_T, _H), lambda bb, ff, be, rt, ba: (0, 0)),
            pl.BlockSpec((1, _FT, _H),
                         lambda bb, ff, be, rt, ba: (be[bb], _fsnake(bb, ff), 0)),
            pl.BlockSpec((1, _FT, _H),
                         lambda bb, ff, be, rt, ba: (be[bb], _fsnake(bb, ff), 0)),
            pl.BlockSpec((1, _H, _FT),
                         lambda bb, ff, be, rt, ba: (be[bb], 0, _fsnake(bb, ff))),
            pl.BlockSpec((_BLK, 1), lambda bb, ff, be, rt, ba: (bb, 0)),
        ],
        out_specs=pl.BlockSpec((_BLK, _H), lambda bb, ff, be, rt, ba: (bb, 0)),
        scratch_shapes=[
            pltpu.VMEM((_BLK, _H), jnp.float32),
            pltpu.VMEM((_BLK, _H), jnp.float32),
        ],
    )
    ys = pl.pallas_call(
        _ffn_kernel,
        grid_spec=grid_spec,
        out_shape=jax.ShapeDtypeStruct((_PP, _H), jnp.float32),
        compiler_params=pltpu.CompilerParams(
            dimension_semantics=("arbitrary", "arbitrary"),
        ),
    )(block_expert, row_token, block_active,
      hs, gate_w, up_w, down_w, row_weight)

    mesh = plsc.VectorSubcoreMesh(core_axis_name="c", subcore_axis_name="s",
                                  num_cores=_NC, num_subcores=_NS)
    out = pl.kernel(
        _sc_combine_body,
        mesh=mesh,
        out_type=jax.ShapeDtypeStruct((_T, _H), jnp.float32),
        scratch_types=[
            pltpu.VMEM((_CTOK,), jnp.int32),
            pltpu.VMEM((_CTOK,), jnp.int32),
            pltpu.VMEM((_CCH, _H), jnp.float32),
            pltpu.VMEM((_CCH, _H), jnp.float32),
            pltpu.VMEM((_CCH, _H), jnp.float32),
            pltpu.VMEM((_CCH, _H), jnp.float32),
            pltpu.SemaphoreType.DMA,
        ],
    )(ys, pos0, pos1)

    return out.reshape(b, s, h), logits


# speedup vs baseline: 1.0989x; 1.0009x over previous
import jax
import jax.numpy as jnp
from jax import lax
from jax.experimental import pallas as pl
from jax.experimental.pallas import tpu as pltpu
from jax.experimental.pallas import tpu_sc as plsc

_B, _S, _H, _F, _E, _K = 1, 2048, 1024, 2816, 8, 2
_T = _B * _S
_P = _T * _K
_BLK = 256
_NB = _P // _BLK + _E
_PP = _NB * _BLK
_FT = 1408
_NF = _F // _FT

_NC, _NS = 2, 16
_NW = _NC * _NS
_CTOK = _T // _NW
_CCH = 16
_NCH = _CTOK // _CCH


def _router_kernel(hs_ref, rw_ref, logits_ref, w_ref, e_ref):
    hs = hs_ref[...]
    logits = jax.lax.dot_general(hs, rw_ref[...], (((1,), (1,)), ((), ())),
                                 preferred_element_type=jnp.float32)
    logits_ref[...] = logits
    m = jnp.max(logits, axis=1, keepdims=True)
    ex = jnp.exp(logits - m)
    sm = ex / jnp.sum(ex, axis=1, keepdims=True)
    iota = jax.lax.broadcasted_iota(jnp.int32, sm.shape, 1)
    m1 = jnp.max(sm, axis=1, keepdims=True)
    a1 = jnp.min(jnp.where(sm == m1, iota, _E), axis=1, keepdims=True)
    sm2 = jnp.where(iota == a1, -jnp.inf, sm)
    m2 = jnp.max(sm2, axis=1, keepdims=True)
    a2 = jnp.min(jnp.where(sm2 == m2, iota, _E), axis=1, keepdims=True)
    w_ref[...] = jnp.concatenate([m1, m2], axis=1)
    e_ref[...] = jnp.concatenate([a1, a2], axis=1)


def _sc_combine_body(ys_ref, p0_ref, p1_ref, out_ref, i0_v, i1_v,
                     r0a, r1a, r0b, r1b, sem):
    wid = lax.axis_index("s") * _NC + lax.axis_index("c")
    base = wid * _CTOK
    pltpu.sync_copy(p0_ref.at[pl.ds(base, _CTOK)], i0_v)
    pltpu.sync_copy(p1_ref.at[pl.ds(base, _CTOK)], i1_v)
    bufs = [(r0a, r1a), (r0b, r1b)]
    pend = [None, None]

    def start(ch):
        r0, r1 = bufs[ch % 2]
        c0 = pltpu.async_copy(ys_ref.at[i0_v.at[pl.ds(ch * _CCH, _CCH)]],
                              r0, sem)
        c1 = pltpu.async_copy(ys_ref.at[i1_v.at[pl.ds(ch * _CCH, _CCH)]],
                              r1, sem)
        pend[ch % 2] = (c0, c1)

    start(0)
    for ch in range(_NCH):
        if ch + 1 < _NCH:
            start(ch + 1)
        c0, c1 = pend[ch % 2]
        c0.wait()
        c1.wait()
        r0, r1 = bufs[ch % 2]

        def add_row(i, c):
            def add16(j, c2):
                sl = pl.ds(j * 16, 16)
                r0[i, sl] = r0[i, sl] + r1[i, sl]
                return c2
            lax.fori_loop(0, _H // 16, add16, 0, unroll=8)
            return c
        lax.fori_loop(0, _CCH, add_row, 0)
        pltpu.sync_copy(r0, out_ref.at[pl.ds(base + ch * _CCH, _CCH)])


def _ffn_kernel(be_ref, rt_ref, ba_ref, hs_ref, g_ref, u_ref, d_ref, w_ref,
                ys_ref, x_s, acc):
    b = pl.program_id(0)
    f = pl.program_id(1)
    active = ba_ref[b] == 1

    @pl.when(active)
    def _():
        @pl.when(f == 0)
        def _():
            def gather(i, c):
                t = rt_ref[b * _BLK + i]
                x_s[pl.ds(i, 1), :] = hs_ref[pl.ds(t, 1), :]
                return c
            jax.lax.fori_loop(0, _BLK, gather, 0, unroll=16)

        x = x_s[...]
        g = jax.lax.dot_general(x, g_ref[0], (((1,), (1,)), ((), ())),
                                preferred_element_type=jnp.float32)
        u = jax.lax.dot_general(x, u_ref[0], (((1,), (1,)), ((), ())),
                                preferred_element_type=jnp.float32)
        h = (g * jax.nn.sigmoid(g)) * u
        part = jax.lax.dot_general(h, d_ref[0], (((1,), (1,)), ((), ())),
                                   preferred_element_type=jnp.float32)

        @pl.when(f == 0)
        def _():
            acc[...] = part

        @pl.when(f != 0)
        def _():
            acc[...] += part

        @pl.when(f == _NF - 1)
        def _():
            ys_ref[...] = acc[...] * w_ref[...]


def kernel(hidden_states, router_w, gate_w, up_w, down_w):
    b, s, h = hidden_states.shape
    hs = hidden_states.reshape(-1, h)

    logits, tw, te = pl.pallas_call(
        _router_kernel,
        out_shape=[
            jax.ShapeDtypeStruct((_T, _E), jnp.float32),
            jax.ShapeDtypeStruct((_T, _K), jnp.float32),
            jax.ShapeDtypeStruct((_T, _K), jnp.int32),
        ],
    )(hs, router_w)

    es = te.reshape(-1)
    ws = tw.reshape(-1)
    onehot = (es[:, None] == jnp.arange(_E)[None, :]).astype(jnp.int32)
    within = jnp.cumsum(onehot, axis=0) - onehot
    rank = jnp.sum(within * onehot, axis=1)
    counts = jnp.sum(onehot, axis=0)
    padded = ((counts + _BLK - 1) // _BLK) * _BLK
    pend = jnp.cumsum(padded)
    poff = pend - padded
    pos = poff[es] + rank
    row_token = jnp.zeros((_PP,), jnp.int32).at[pos].set(
        jnp.arange(_P, dtype=jnp.int32) // _K)
    row_weight = jnp.zeros((_PP, 1), jnp.float32).at[pos, 0].set(ws)
    pos2 = pos.reshape(_T, _K)
    pos0 = pos2[:, 0].astype(jnp.int32)
    pos1 = pos2[:, 1].astype(jnp.int32)
    starts = jnp.arange(_NB, dtype=jnp.int32) * _BLK
    total = pend[_E - 1]
    block_active = (starts < total).astype(jnp.int32)
    starts_c = jnp.minimum(starts, total - 1)
    block_expert = jnp.searchsorted(pend, starts_c, side='right').astype(jnp.int32)

    def _fsnake(bb, ff):
        return jnp.where(bb % 2 == 0, ff, _NF - 1 - ff)

    grid_spec = pltpu.PrefetchScalarGridSpec(
        num_scalar_prefetch=3,
        grid=(_NB, _NF),
        in_specs=[
            pl.BlockSpec((_T, _H), lambda bb, ff, be, rt, ba: (0, 0)),
            pl.BlockSpec((1, _FT, _H),
                         lambda bb, ff, be, rt, ba: (be[bb], _fsnake(bb, ff), 0)),
            pl.BlockSpec((1, _FT, _H),
                         lambda bb, ff, be, rt, ba: (be[bb], _fsnake(bb, ff), 0)),
            pl.BlockSpec((1, _H, _FT),
                         lambda bb, ff, be, rt, ba: (be[bb], 0, _fsnake(bb, ff))),
            pl.BlockSpec((_BLK, 1), lambda bb, ff, be, rt, ba: (bb, 0)),
        ],
        out_specs=pl.BlockSpec((_BLK, _H), lambda bb, ff, be, rt, ba: (bb, 0)),
        scratch_shapes=[
            pltpu.VMEM((_BLK, _H), jnp.float32),
            pltpu.VMEM((_BLK, _H), jnp.float32),
        ],
    )
    ys = pl.pallas_call(
        _ffn_kernel,
        grid_spec=grid_spec,
        out_shape=jax.ShapeDtypeStruct((_PP, _H), jnp.float32),
        compiler_params=pltpu.CompilerParams(
            dimension_semantics=("arbitrary", "arbitrary"),
        ),
    )(block_expert, row_token, block_active,
      hs, gate_w, up_w, down_w, row_weight)

    mesh = plsc.VectorSubcoreMesh(core_axis_name="c", subcore_axis_name="s",
                                  num_cores=_NC, num_subcores=_NS)
    out = pl.kernel(
        _sc_combine_body,
        mesh=mesh,
        out_type=jax.ShapeDtypeStruct((_T, _H), jnp.float32),
        scratch_types=[
            pltpu.VMEM((_CTOK,), jnp.int32),
            pltpu.VMEM((_CTOK,), jnp.int32),
            pltpu.VMEM((_CCH, _H), jnp.float32),
            pltpu.VMEM((_CCH, _H), jnp.float32),
            pltpu.VMEM((_CCH, _H), jnp.float32),
            pltpu.VMEM((_CCH, _H), jnp.float32),
            pltpu.SemaphoreType.DMA,
        ],
    )(ys, pos0, pos1)

    return out.reshape(b, s, h), logits


# speedup vs baseline: 1.1651x; 1.0603x over previous
import jax
import jax.numpy as jnp
from jax import lax
from jax.experimental import pallas as pl
from jax.experimental.pallas import tpu as pltpu
from jax.experimental.pallas import tpu_sc as plsc

_B, _S, _H, _F, _E, _K = 1, 2048, 1024, 2816, 8, 2
_T = _B * _S
_P = _T * _K
_BLK = 256
_NB = _P // _BLK + _E
_PP = _NB * _BLK
_FT = 1408
_NF = _F // _FT

_NC, _NS = 2, 16
_NW = _NC * _NS
_CTOK = _T // _NW
_CCH = 16
_NCH = _CTOK // _CCH


def _router_kernel(hs_ref, rw_ref, logits_ref, w_ref, e_ref):
    hs = hs_ref[...]
    logits = jax.lax.dot_general(hs, rw_ref[...], (((1,), (1,)), ((), ())),
                                 preferred_element_type=jnp.float32)
    logits_ref[...] = logits
    m = jnp.max(logits, axis=1, keepdims=True)
    ex = jnp.exp(logits - m)
    sm = ex / jnp.sum(ex, axis=1, keepdims=True)
    iota = jax.lax.broadcasted_iota(jnp.int32, sm.shape, 1)
    m1 = jnp.max(sm, axis=1, keepdims=True)
    a1 = jnp.min(jnp.where(sm == m1, iota, _E), axis=1, keepdims=True)
    sm2 = jnp.where(iota == a1, -jnp.inf, sm)
    m2 = jnp.max(sm2, axis=1, keepdims=True)
    a2 = jnp.min(jnp.where(sm2 == m2, iota, _E), axis=1, keepdims=True)
    w_ref[...] = jnp.concatenate([m1, m2], axis=1)
    e_ref[...] = jnp.concatenate([a1, a2], axis=1)


def _sc_combine_body(ys_ref, p0_ref, p1_ref, out_ref, i0_v, i1_v,
                     r0a, r1a, r0b, r1b, sem):
    wid = lax.axis_index("s") * _NC + lax.axis_index("c")
    base = wid * _CTOK
    pltpu.sync_copy(p0_ref.at[pl.ds(base, _CTOK)], i0_v)
    pltpu.sync_copy(p1_ref.at[pl.ds(base, _CTOK)], i1_v)
    bufs = [(r0a, r1a), (r0b, r1b)]
    pend = [None, None]

    def start(ch):
        r0, r1 = bufs[ch % 2]
        c0 = pltpu.async_copy(ys_ref.at[i0_v.at[pl.ds(ch * _CCH, _CCH)]],
                              r0, sem)
        c1 = pltpu.async_copy(ys_ref.at[i1_v.at[pl.ds(ch * _CCH, _CCH)]],
                              r1, sem)
        pend[ch % 2] = (c0, c1)

    start(0)
    for ch in range(_NCH):
        if ch + 1 < _NCH:
            start(ch + 1)
        c0, c1 = pend[ch % 2]
        c0.wait()
        c1.wait()
        r0, r1 = bufs[ch % 2]

        def add_row(i, c):
            def add16(j, c2):
                sl = pl.ds(j * 16, 16)
                r0[i, sl] = r0[i, sl] + r1[i, sl]
                return c2
            lax.fori_loop(0, _H // 16, add16, 0, unroll=8)
            return c
        lax.fori_loop(0, _CCH, add_row, 0)
        pltpu.sync_copy(r0, out_ref.at[pl.ds(base + ch * _CCH, _CCH)])


def _ffn_kernel(be_ref, rt_ref, ba_ref, hs_ref, g_ref, u_ref, d_ref, w_ref,
                ys_ref, x_s, acc):
    b = pl.program_id(0)
    f = pl.program_id(1)
    active = ba_ref[b] == 1

    @pl.when(active)
    def _():
        @pl.when(f == 0)
        def _():
            def gather(i, c):
                t = rt_ref[b * _BLK + i]
                x_s[pl.ds(i, 1), :] = hs_ref[pl.ds(t, 1), :]
                return c
            jax.lax.fori_loop(0, _BLK, gather, 0, unroll=16)

        x = x_s[...]
        g = jax.lax.dot_general(x, g_ref[0], (((1,), (1,)), ((), ())),
                                preferred_element_type=jnp.float32)
        u = jax.lax.dot_general(x, u_ref[0], (((1,), (1,)), ((), ())),
                                preferred_element_type=jnp.float32)
        h = (g * jax.nn.sigmoid(g)) * u
        part = jax.lax.dot_general(h, d_ref[0], (((1,), (1,)), ((), ())),
                                   preferred_element_type=jnp.float32)

        @pl.when(f == 0)
        def _():
            acc[...] = part

        @pl.when(f != 0)
        def _():
            acc[...] += part

        @pl.when(f == _NF - 1)
        def _():
            ys_ref[...] = acc[...] * w_ref[...]


def kernel(hidden_states, router_w, gate_w, up_w, down_w):
    b, s, h = hidden_states.shape
    hs = hidden_states.reshape(-1, h)

    logits, tw, te = pl.pallas_call(
        _router_kernel,
        out_shape=[
            jax.ShapeDtypeStruct((_T, _E), jnp.float32),
            jax.ShapeDtypeStruct((_T, _K), jnp.float32),
            jax.ShapeDtypeStruct((_T, _K), jnp.int32),
        ],
    )(hs, router_w)

    es = te.reshape(-1)
    ws = tw.reshape(-1)
    onehot = (es[:, None] == jnp.arange(_E)[None, :]).astype(jnp.int32)
    within = jnp.cumsum(onehot, axis=0) - onehot
    rank = jnp.sum(within * onehot, axis=1)
    counts = jnp.sum(onehot, axis=0)
    padded = ((counts + _BLK - 1) // _BLK) * _BLK
    pend = jnp.cumsum(padded)
    poff = pend - padded
    pos = poff[es] + rank
    tokf = (jnp.arange(_P, dtype=jnp.int32) // _K).astype(jnp.float32)
    row_pair = jnp.zeros((_PP, 2), jnp.float32).at[pos].set(
        jnp.stack([tokf, ws], axis=1))
    row_token = row_pair[:, 0].astype(jnp.int32)
    row_weight = row_pair[:, 1:2]
    pos2 = pos.reshape(_T, _K)
    pos0 = pos2[:, 0].astype(jnp.int32)
    pos1 = pos2[:, 1].astype(jnp.int32)
    starts = jnp.arange(_NB, dtype=jnp.int32) * _BLK
    total = pend[_E - 1]
    block_active = (starts < total).astype(jnp.int32)
    starts_c = jnp.minimum(starts, total - 1)
    block_expert = jnp.sum(
        (pend[None, :] <= starts_c[:, None]).astype(jnp.int32), axis=1)

    def _fsnake(bb, ff):
        return jnp.where(bb % 2 == 0, ff, _NF - 1 - ff)

    grid_spec = pltpu.PrefetchScalarGridSpec(
        num_scalar_prefetch=3,
        grid=(_NB, _NF),
        in_specs=[
            pl.BlockSpec((_T, _H), lambda bb, ff, be, rt, ba: (0, 0)),
            pl.BlockSpec((1, _FT, _H),
                         lambda bb, ff, be, rt, ba: (be[bb], _fsnake(bb, ff), 0)),
            pl.BlockSpec((1, _FT, _H),
                         lambda bb, ff, be, rt, ba: (be[bb], _fsnake(bb, ff), 0)),
            pl.BlockSpec((1, _H, _FT),
                         lambda bb, ff, be, rt, ba: (be[bb], 0, _fsnake(bb, ff))),
            pl.BlockSpec((_BLK, 1), lambda bb, ff, be, rt, ba: (bb, 0)),
        ],
        out_specs=pl.BlockSpec((_BLK, _H), lambda bb, ff, be, rt, ba: (bb, 0)),
        scratch_shapes=[
            pltpu.VMEM((_BLK, _H), jnp.float32),
            pltpu.VMEM((_BLK, _H), jnp.float32),
        ],
    )
    ys = pl.pallas_call(
        _ffn_kernel,
        grid_spec=grid_spec,
        out_shape=jax.ShapeDtypeStruct((_PP, _H), jnp.float32),
        compiler_params=pltpu.CompilerParams(
            dimension_semantics=("arbitrary", "arbitrary"),
        ),
    )(block_expert, row_token, block_active,
      hs, gate_w, up_w, down_w, row_weight)

    mesh = plsc.VectorSubcoreMesh(core_axis_name="c", subcore_axis_name="s",
                                  num_cores=_NC, num_subcores=_NS)
    out = pl.kernel(
        _sc_combine_body,
        mesh=mesh,
        out_type=jax.ShapeDtypeStruct((_T, _H), jnp.float32),
        scratch_types=[
            pltpu.VMEM((_CTOK,), jnp.int32),
            pltpu.VMEM((_CTOK,), jnp.int32),
            pltpu.VMEM((_CCH, _H), jnp.float32),
            pltpu.VMEM((_CCH, _H), jnp.float32),
            pltpu.VMEM((_CCH, _H), jnp.float32),
            pltpu.VMEM((_CCH, _H), jnp.float32),
            pltpu.SemaphoreType.DMA,
        ],
    )(ys, pos0, pos1)

    return out.reshape(b, s, h), logits


# speedup vs baseline: 1.1848x; 1.0169x over previous
import jax
import jax.numpy as jnp
from jax import lax
from jax.experimental import pallas as pl
from jax.experimental.pallas import tpu as pltpu
from jax.experimental.pallas import tpu_sc as plsc

_B, _S, _H, _F, _E, _K = 1, 2048, 1024, 2816, 8, 2
_T = _B * _S
_P = _T * _K
_BLK = 256
_NB = _P // _BLK + _E
_PP = _NB * _BLK
_FT = 1408
_NF = _F // _FT

_NC, _NS = 2, 16
_NW = _NC * _NS
_CTOK = _T // _NW
_CCH = 16
_NCH = _CTOK // _CCH


def _router_kernel(hs_ref, rw_ref, logits_ref, w_ref, e_ref):
    hs = hs_ref[...]
    logits = jax.lax.dot_general(hs, rw_ref[...], (((1,), (1,)), ((), ())),
                                 preferred_element_type=jnp.float32)
    logits_ref[...] = logits
    m = jnp.max(logits, axis=1, keepdims=True)
    ex = jnp.exp(logits - m)
    sm = ex / jnp.sum(ex, axis=1, keepdims=True)
    iota = jax.lax.broadcasted_iota(jnp.int32, sm.shape, 1)
    m1 = jnp.max(sm, axis=1, keepdims=True)
    a1 = jnp.min(jnp.where(sm == m1, iota, _E), axis=1, keepdims=True)
    sm2 = jnp.where(iota == a1, -jnp.inf, sm)
    m2 = jnp.max(sm2, axis=1, keepdims=True)
    a2 = jnp.min(jnp.where(sm2 == m2, iota, _E), axis=1, keepdims=True)
    w_ref[...] = jnp.concatenate([m1, m2], axis=1)
    e_ref[...] = jnp.concatenate([a1, a2], axis=1)


def _sc_combine_body(ys_ref, p0_ref, p1_ref, out_ref, i0_v, i1_v,
                     r0a, r1a, r0b, r1b, sem):
    wid = lax.axis_index("s") * _NC + lax.axis_index("c")
    base = wid * _CTOK
    pltpu.sync_copy(p0_ref.at[pl.ds(base, _CTOK)], i0_v)
    pltpu.sync_copy(p1_ref.at[pl.ds(base, _CTOK)], i1_v)
    bufs = [(r0a, r1a), (r0b, r1b)]
    pend = [None, None]

    def start(ch):
        r0, r1 = bufs[ch % 2]
        c0 = pltpu.async_copy(ys_ref.at[i0_v.at[pl.ds(ch * _CCH, _CCH)]],
                              r0, sem)
        c1 = pltpu.async_copy(ys_ref.at[i1_v.at[pl.ds(ch * _CCH, _CCH)]],
                              r1, sem)
        pend[ch % 2] = (c0, c1)

    start(0)
    for ch in range(_NCH):
        if ch + 1 < _NCH:
            start(ch + 1)
        c0, c1 = pend[ch % 2]
        c0.wait()
        c1.wait()
        r0, r1 = bufs[ch % 2]

        def add_row(i, c):
            def add16(j, c2):
                sl = pl.ds(j * 16, 16)
                r0[i, sl] = r0[i, sl] + r1[i, sl]
                return c2
            lax.fori_loop(0, _H // 16, add16, 0, unroll=8)
            return c
        lax.fori_loop(0, _CCH, add_row, 0)
        pltpu.sync_copy(r0, out_ref.at[pl.ds(base + ch * _CCH, _CCH)])


def _ffn_kernel(be_ref, rt_ref, ba_ref, hs_ref, g_ref, u_ref, d_ref, w_ref,
                ys_ref, x_s, acc):
    b = pl.program_id(0)
    f = pl.program_id(1)
    active = ba_ref[b] == 1

    @pl.when(active)
    def _():
        @pl.when(f == 0)
        def _():
            def gather(i, c):
                t = rt_ref[b * _BLK + i]
                x_s[pl.ds(i, 1), :] = hs_ref[pl.ds(t, 1), :]
                return c
            jax.lax.fori_loop(0, _BLK, gather, 0, unroll=16)

        x = x_s[...]
        g = jax.lax.dot_general(x, g_ref[0], (((1,), (1,)), ((), ())),
                                preferred_element_type=jnp.float32)
        u = jax.lax.dot_general(x, u_ref[0], (((1,), (1,)), ((), ())),
                                preferred_element_type=jnp.float32)
        h = (g * jax.nn.sigmoid(g)) * u
        part = jax.lax.dot_general(h, d_ref[0], (((1,), (1,)), ((), ())),
                                   preferred_element_type=jnp.float32)

        @pl.when(f == 0)
        def _():
            acc[...] = part

        @pl.when(f != 0)
        def _():
            acc[...] += part

        @pl.when(f == _NF - 1)
        def _():
            ys_ref[...] = acc[...] * w_ref[...]


def kernel(hidden_states, router_w, gate_w, up_w, down_w):
    b, s, h = hidden_states.shape
    hs = hidden_states.reshape(-1, h)

    logits, tw, te = pl.pallas_call(
        _router_kernel,
        out_shape=[
            jax.ShapeDtypeStruct((_T, _E), jnp.float32),
            jax.ShapeDtypeStruct((_T, _K), jnp.float32),
            jax.ShapeDtypeStruct((_T, _K), jnp.int32),
        ],
    )(hs, router_w)

    es = te.reshape(-1)
    ws = tw.reshape(-1)
    onehot = (es[:, None] == jnp.arange(_E)[None, :]).astype(jnp.float32)
    cs = jnp.cumsum(onehot, axis=0)
    counts = cs[-1].astype(jnp.int32)
    padded = ((counts + _BLK - 1) // _BLK) * _BLK
    pend = jnp.cumsum(padded)
    poff = pend - padded
    pos = (jnp.sum(onehot * (cs + poff.astype(jnp.float32)[None, :]), axis=1)
           - 1.0).astype(jnp.int32)
    tokf = (jnp.arange(_P, dtype=jnp.int32) // _K).astype(jnp.float32)
    row_pair = jnp.zeros((_PP, 2), jnp.float32).at[pos].set(
        jnp.stack([tokf, ws], axis=1))
    row_token = row_pair[:, 0].astype(jnp.int32)
    row_weight = row_pair[:, 1:2]
    pos2 = pos.reshape(_T, _K)
    pos0 = pos2[:, 0].astype(jnp.int32)
    pos1 = pos2[:, 1].astype(jnp.int32)
    starts = jnp.arange(_NB, dtype=jnp.int32) * _BLK
    total = pend[_E - 1]
    block_active = (starts < total).astype(jnp.int32)
    starts_c = jnp.minimum(starts, total - 1)
    block_expert = jnp.sum(
        (pend[None, :] <= starts_c[:, None]).astype(jnp.int32), axis=1)

    def _fsnake(bb, ff):
        return jnp.where(bb % 2 == 0, ff, _NF - 1 - ff)

    grid_spec = pltpu.PrefetchScalarGridSpec(
        num_scalar_prefetch=3,
        grid=(_NB, _NF),
        in_specs=[
            pl.BlockSpec((_T, _H), lambda bb, ff, be, rt, ba: (0, 0)),
            pl.BlockSpec((1, _FT, _H),
                         lambda bb, ff, be, rt, ba: (be[bb], _fsnake(bb, ff), 0)),
            pl.BlockSpec((1, _FT, _H),
                         lambda bb, ff, be, rt, ba: (be[bb], _fsnake(bb, ff), 0)),
            pl.BlockSpec((1, _H, _FT),
                         lambda bb, ff, be, rt, ba: (be[bb], 0, _fsnake(bb, ff))),
            pl.BlockSpec((_BLK, 1), lambda bb, ff, be, rt, ba: (bb, 0)),
        ],
        out_specs=pl.BlockSpec((_BLK, _H), lambda bb, ff, be, rt, ba: (bb, 0)),
        scratch_shapes=[
            pltpu.VMEM((_BLK, _H), jnp.float32),
            pltpu.VMEM((_BLK, _H), jnp.float32),
        ],
    )
    ys = pl.pallas_call(
        _ffn_kernel,
        grid_spec=grid_spec,
        out_shape=jax.ShapeDtypeStruct((_PP, _H), jnp.float32),
        compiler_params=pltpu.CompilerParams(
            dimension_semantics=("arbitrary", "arbitrary"),
        ),
    )(block_expert, row_token, block_active,
      hs, gate_w, up_w, down_w, row_weight)

    mesh = plsc.VectorSubcoreMesh(core_axis_name="c", subcore_axis_name="s",
                                  num_cores=_NC, num_subcores=_NS)
    out = pl.kernel(
        _sc_combine_body,
        mesh=mesh,
        out_type=jax.ShapeDtypeStruct((_T, _H), jnp.float32),
        scratch_types=[
            pltpu.VMEM((_CTOK,), jnp.int32),
            pltpu.VMEM((_CTOK,), jnp.int32),
            pltpu.VMEM((_CCH, _H), jnp.float32),
            pltpu.VMEM((_CCH, _H), jnp.float32),
            pltpu.VMEM((_CCH, _H), jnp.float32),
            pltpu.VMEM((_CCH, _H), jnp.float32),
            pltpu.SemaphoreType.DMA,
        ],
    )(ys, pos0, pos1)

    return out.reshape(b, s, h), logits
